# baseline (speedup 1.0000x reference)
import functools

import jax
import jax.numpy as jnp
from jax import lax
from jax.experimental import pallas as pl
from jax.experimental.pallas import tpu as pltpu

B, N, G, EMB, H = 2, 32768, 512, 256, 1024
BG = B * G
NTOT = B * N
P = 2048
NT = NTOT // P
TPB = N // P


NCOPY = 4


def _scatter_stage(idx_sm, xyz_ref, feats_ref, idxv_ref, centers_ref,
                   w1_ref, out_ref, *scratch):
    accs, pk = scratch[:NCOPY], scratch[NCOPY]
    i = pl.program_id(0)
    b = i // TPB

    loc = idxv_ref[...] - b * G
    ohT = (lax.broadcasted_iota(jnp.int32, (G, P), 0)
           == loc).astype(jnp.float32)
    cgT = jnp.dot(centers_ref[0], ohT,
                  preferred_element_type=jnp.float32)
    nbhT = xyz_ref[...] - cgT
    d2 = jnp.sum(nbhT * nbhT, axis=0, keepdims=True)
    r = lax.rsqrt(d2 + 1e-30)
    distT = d2 * r
    nbhnT = nbhT * r

    nfT = jnp.concatenate(
        [feats_ref[...], nbhnT, distT, jnp.zeros((3, P), jnp.float32)], axis=0)
    feat = lax.dot_general(nfT, w1_ref[...], (((0,), (0,)), ((), ())),
                           preferred_element_type=jnp.float32)
    pk[...] = feat

    @pl.when(i == 0)
    def _():
        neg = jnp.full((BG, 8, 128), -jnp.inf, jnp.float32)
        for ar in accs:
            ar[...] = neg

    base = i * P

    def body(j, carry):
        p = j * 8
        pkb = pk[pl.ds(p, 8), :].reshape(8, 8, 128)
        for c in range(8):
            ar = accs[c % NCOPY]
            g = idx_sm[base + p + c]
            ar[g] = jnp.maximum(ar[g], pkb[c])
        return carry

    lax.fori_loop(0, P // 8, body, 0, unroll=16)

    @pl.when(i == NT - 1)
    def _():
        m = accs[0][...]
        for ar in accs[1:]:
            m = jnp.maximum(m, ar[...])
        out_ref[...] = m


def _ln(x, g, b):
    m = jnp.mean(x, axis=-1, keepdims=True)
    v = jnp.mean((x - m) * (x - m), axis=-1, keepdims=True)
    return (x - m) * lax.rsqrt(v + 1e-5) * g + b


def _gelu(x):
    return 0.5 * x * (1.0 + lax.erf(x * 0.7071067811865476))


def _mlp_stage(agg_ref, b1_ref, win_ref, bin_ref, g0_ref, be0_ref,
               wr_ref, br_ref, gr_ref, betar_ref, wout_ref, bout_ref,
               out_ref):
    x = jnp.maximum(agg_ref[...] + b1_ref[...], 0.0)
    h = jnp.dot(x, win_ref[...], preferred_element_type=jnp.float32)
    h = _gelu(_ln(h + bin_ref[...], g0_ref[...], be0_ref[...]))
    for l in range(wr_ref.shape[0]):
        t = jnp.dot(h, wr_ref[l], preferred_element_type=jnp.float32)
        t = _ln(t + br_ref[pl.ds(l, 1), :], gr_ref[pl.ds(l, 1), :],
                betar_ref[pl.ds(l, 1), :])
        h = h + _gelu(t)
    out_ref[...] = jnp.dot(h, wout_ref[...],
                           preferred_element_type=jnp.float32) + bout_ref[...]


@functools.partial(jax.jit, static_argnames=())
def kernel(xyz, centers, masks, idx, mask_batch, W1, b1, Win, bin_, g0, be0,
           Wr, br, gr, betar, Wout, bout):
    xyz_t = xyz.reshape(NTOT, 3).T
    feats = (masks.reshape(1, NTOT)
             * jnp.asarray(mask_batch).astype(masks.dtype))
    idx_i = idx.astype(jnp.int32)
    idx_v = idx_i.reshape(1, NTOT)
    centers_t = centers.transpose(0, 2, 1)
    w1p = jnp.concatenate([W1, jnp.zeros((3, H), W1.dtype)], axis=0)

    grid_spec = pltpu.PrefetchScalarGridSpec(
        num_scalar_prefetch=1,
        grid=(NT,),
        in_specs=[
            pl.BlockSpec((3, P), lambda i, s: (0, i)),
            pl.BlockSpec((1, P), lambda i, s: (0, i)),
            pl.BlockSpec((1, P), lambda i, s: (0, i)),
            pl.BlockSpec((1, 3, G), lambda i, s: (i // TPB, 0, 0)),
            pl.BlockSpec((8, H), lambda i, s: (0, 0)),
        ],
        out_specs=pl.BlockSpec((BG, 8, 128), lambda i, s: (0, 0, 0)),
        scratch_shapes=[pltpu.VMEM((BG, 8, 128), jnp.float32)
                        for _ in range(NCOPY)]
        + [pltpu.VMEM((P, H), jnp.float32)],
    )
    agg = pl.pallas_call(
        _scatter_stage,
        grid_spec=grid_spec,
        out_shape=jax.ShapeDtypeStruct((BG, 8, 128), jnp.float32),
        compiler_params=pltpu.CompilerParams(
            dimension_semantics=("arbitrary",)),
    )(idx_i, xyz_t, feats, idx_v, centers_t, w1p)

    out = pl.pallas_call(
        _mlp_stage,
        out_shape=jax.ShapeDtypeStruct((BG, EMB), jnp.float32),
    )(agg.reshape(BG, H), b1.reshape(1, H), Win, bin_.reshape(1, H),
      g0.reshape(1, H), be0.reshape(1, H), Wr, br, gr, betar, Wout,
      bout.reshape(1, EMB))

    return out.reshape(B, G, EMB)


# speedup vs baseline: 1.0452x; 1.0452x over previous
import functools

import jax
import jax.numpy as jnp
from jax import lax
from jax.experimental import pallas as pl
from jax.experimental.pallas import tpu as pltpu

B, N, G, EMB, H = 2, 32768, 512, 256, 1024
NUM_RES = 3
BG = B * G
NTOT = B * N
P = 2048
NT = NTOT // P
TPB = N // P


NCOPY = 4


def _scatter_stage(idx_sm, xyz_ref, feats_ref, idxv_ref, centers_ref,
                   w1_ref, b1_ref, win_ref, bin_ref, g0_ref, be0_ref,
                   wr_ref, br_ref, gr_ref, betar_ref, wout_ref, bout_ref,
                   out_ref, *scratch):
    accs, pk = scratch[:NCOPY], scratch[NCOPY]
    i = pl.program_id(0)
    b = i // TPB

    loc = idxv_ref[...] - b * G
    ohT = (lax.broadcasted_iota(jnp.int32, (G, P), 0)
           == loc).astype(jnp.float32)
    cgT = jnp.dot(centers_ref[0], ohT,
                  preferred_element_type=jnp.float32)
    nbhT = xyz_ref[...] - cgT
    d2 = jnp.sum(nbhT * nbhT, axis=0, keepdims=True)
    r = lax.rsqrt(d2 + 1e-30)
    distT = d2 * r
    nbhnT = nbhT * r

    nfT = jnp.concatenate(
        [feats_ref[...], nbhnT, distT, jnp.zeros((3, P), jnp.float32)], axis=0)
    feat = lax.dot_general(nfT, w1_ref[...], (((0,), (0,)), ((), ())),
                           preferred_element_type=jnp.float32)
    pk[...] = feat

    @pl.when(i == 0)
    def _():
        neg = jnp.full((BG, 8, 128), -jnp.inf, jnp.float32)
        for ar in accs:
            ar[...] = neg

    base = i * P

    def body(j, carry):
        p = j * 8
        pkb = pk[pl.ds(p, 8), :].reshape(8, 8, 128)
        for c in range(8):
            ar = accs[c % NCOPY]
            g = idx_sm[base + p + c]
            ar[g] = jnp.maximum(ar[g], pkb[c])
        return carry

    lax.fori_loop(0, P // 8, body, 0, unroll=16)

    @pl.when(i == NT - 1)
    def _():
        m = accs[0][...]
        for ar in accs[1:]:
            m = jnp.maximum(m, ar[...])
        agg = m.reshape(BG, H)
        x = jnp.maximum(agg + b1_ref[...], 0.0)
        h = jnp.dot(x, win_ref[...], preferred_element_type=jnp.float32)
        h = _gelu(_ln(h + bin_ref[...], g0_ref[...], be0_ref[...]))
        for l in range(wr_ref.shape[0]):
            t = jnp.dot(h, wr_ref[l], preferred_element_type=jnp.float32)
            t = _ln(t + br_ref[pl.ds(l, 1), :], gr_ref[pl.ds(l, 1), :],
                    betar_ref[pl.ds(l, 1), :])
            h = h + _gelu(t)
        out_ref[...] = jnp.dot(h, wout_ref[...],
                               preferred_element_type=jnp.float32
                               ) + bout_ref[...]


def _ln(x, g, b):
    m = jnp.mean(x, axis=-1, keepdims=True)
    v = jnp.mean((x - m) * (x - m), axis=-1, keepdims=True)
    return (x - m) * lax.rsqrt(v + 1e-5) * g + b


def _gelu(x):
    return 0.5 * x * (1.0 + lax.erf(x * 0.7071067811865476))


@functools.partial(jax.jit, static_argnames=())
def kernel(xyz, centers, masks, idx, mask_batch, W1, b1, Win, bin_, g0, be0,
           Wr, br, gr, betar, Wout, bout):
    xyz_t = xyz.reshape(NTOT, 3).T
    feats = (masks.reshape(1, NTOT)
             * jnp.asarray(mask_batch).astype(masks.dtype))
    idx_i = idx.astype(jnp.int32)
    idx_v = idx_i.reshape(1, NTOT)
    centers_t = centers.transpose(0, 2, 1)
    w1p = jnp.concatenate([W1, jnp.zeros((3, H), W1.dtype)], axis=0)

    grid_spec = pltpu.PrefetchScalarGridSpec(
        num_scalar_prefetch=1,
        grid=(NT,),
        in_specs=[
            pl.BlockSpec((3, P), lambda i, s: (0, i)),
            pl.BlockSpec((1, P), lambda i, s: (0, i)),
            pl.BlockSpec((1, P), lambda i, s: (0, i)),
            pl.BlockSpec((1, 3, G), lambda i, s: (i // TPB, 0, 0)),
            pl.BlockSpec((8, H), lambda i, s: (0, 0)),
            pl.BlockSpec((1, H), lambda i, s: (0, 0)),
            pl.BlockSpec((H, H), lambda i, s: (0, 0)),
            pl.BlockSpec((1, H), lambda i, s: (0, 0)),
            pl.BlockSpec((1, H), lambda i, s: (0, 0)),
            pl.BlockSpec((1, H), lambda i, s: (0, 0)),
            pl.BlockSpec((NUM_RES, H, H), lambda i, s: (0, 0, 0)),
            pl.BlockSpec((NUM_RES, H), lambda i, s: (0, 0)),
            pl.BlockSpec((NUM_RES, H), lambda i, s: (0, 0)),
            pl.BlockSpec((NUM_RES, H), lambda i, s: (0, 0)),
            pl.BlockSpec((H, EMB), lambda i, s: (0, 0)),
            pl.BlockSpec((1, EMB), lambda i, s: (0, 0)),
        ],
        out_specs=pl.BlockSpec((BG, EMB), lambda i, s: (0, 0)),
        scratch_shapes=[pltpu.VMEM((BG, 8, 128), jnp.float32)
                        for _ in range(NCOPY)]
        + [pltpu.VMEM((P, H), jnp.float32)],
    )
    out = pl.pallas_call(
        _scatter_stage,
        grid_spec=grid_spec,
        out_shape=jax.ShapeDtypeStruct((BG, EMB), jnp.float32),
        compiler_params=pltpu.CompilerParams(
            dimension_semantics=("arbitrary",)),
    )(idx_i, xyz_t, feats, idx_v, centers_t, w1p, b1.reshape(1, H), Win,
      bin_.reshape(1, H), g0.reshape(1, H), be0.reshape(1, H), Wr, br, gr,
      betar, Wout, bout.reshape(1, EMB))

    return out.reshape(B, G, EMB)
